# baseline (speedup 1.0000x reference)
import functools

import jax
import jax.numpy as jnp
from jax import lax
from jax.experimental import pallas as pl
from jax.experimental.pallas import tpu as pltpu
from jax.experimental.pallas import tpu_sc as plsc

HID = 768
LANES = 16
CHUNK = 32


def _emb_body(ids_hbm, pos_hbm, tt_hbm, word_hbm, type_hbm, postab_hbm,
              out_hbm, widx, pidx, tidx, wrows, prows, trows,
              sem_w, sem_p, sem_t, *, tokens_per_worker):
    num_cores = 2
    wid = lax.axis_index("s") * num_cores + lax.axis_index("c")
    base = wid * tokens_per_worker

    for c in range(tokens_per_worker // CHUNK):
        off = base + c * CHUNK
        pltpu.sync_copy(ids_hbm.at[pl.ds(off, CHUNK)], widx)
        pltpu.sync_copy(pos_hbm.at[pl.ds(off, CHUNK)], pidx)
        pltpu.sync_copy(tt_hbm.at[pl.ds(off, CHUNK)], tidx)
        cw = pltpu.async_copy(word_hbm.at[widx], wrows, sem_w)
        cp = pltpu.async_copy(postab_hbm.at[pidx], prows, sem_p)
        ct = pltpu.async_copy(type_hbm.at[tidx], trows, sem_t)
        cw.wait()
        cp.wait()
        ct.wait()

        @plsc.parallel_loop(0, CHUNK)
        def tok_body(t):
            for j in range(HID // LANES):
                sl = pl.ds(j * LANES, LANES)
                wrows[t, sl] = wrows[t, sl] + prows[t, sl] + trows[t, sl]

        pltpu.sync_copy(wrows, out_hbm.at[pl.ds(off, CHUNK)])


def kernel(input_ids, position_ids, token_type_ids, word_embeddings,
           token_type_embeddings, position_embeddings):
    b, s = input_ids.shape
    ntok = b * s
    hid = word_embeddings.shape[1]
    info = plsc.get_sparse_core_info()
    nworkers = info.num_cores * info.num_subcores
    tokens_per_worker = ntok // nworkers

    ids = input_ids.reshape(ntok).astype(jnp.int32)
    pos = position_ids.reshape(ntok).astype(jnp.int32)
    tts = token_type_ids.reshape(ntok).astype(jnp.int32)

    run = pl.kernel(
        functools.partial(_emb_body, tokens_per_worker=tokens_per_worker),
        mesh=plsc.VectorSubcoreMesh(core_axis_name="c", subcore_axis_name="s"),
        out_type=jax.ShapeDtypeStruct((ntok, hid), jnp.float32),
        scratch_types=[
            pltpu.VMEM((CHUNK,), jnp.int32),
            pltpu.VMEM((CHUNK,), jnp.int32),
            pltpu.VMEM((CHUNK,), jnp.int32),
            pltpu.VMEM((CHUNK, hid), jnp.float32),
            pltpu.VMEM((CHUNK, hid), jnp.float32),
            pltpu.VMEM((CHUNK, hid), jnp.float32),
            pltpu.SemaphoreType.DMA,
            pltpu.SemaphoreType.DMA,
            pltpu.SemaphoreType.DMA,
        ],
    )
    out = run(ids, pos, tts, word_embeddings, token_type_embeddings,
              position_embeddings)
    return out.reshape(b, s, hid)


# speedup vs baseline: 1.0393x; 1.0393x over previous
import functools

import jax
import jax.numpy as jnp
from jax import lax
from jax.experimental import pallas as pl
from jax.experimental.pallas import tpu as pltpu
from jax.experimental.pallas import tpu_sc as plsc

HID = 768
LANES = 16
CHUNK = 16


def _emb_body(ids_hbm, pos_hbm, tt_hbm, word_hbm, type_hbm, postab_hbm,
              out_hbm, widx, pidx, tidx, wrows, prows, trows,
              sem_w, sem_p, sem_t, sem_o, *, tokens_per_worker):
    num_cores = 2
    wid = lax.axis_index("s") * num_cores + lax.axis_index("c")
    base = wid * tokens_per_worker
    nch = tokens_per_worker // CHUNK

    def fire(c, b):
        off = base + c * CHUNK
        pltpu.sync_copy(ids_hbm.at[pl.ds(off, CHUNK)], widx[b])
        pltpu.sync_copy(pos_hbm.at[pl.ds(off, CHUNK)], pidx[b])
        pltpu.sync_copy(tt_hbm.at[pl.ds(off, CHUNK)], tidx[b])
        pltpu.async_copy(word_hbm.at[widx[b]], wrows[b], sem_w[b])
        pltpu.async_copy(postab_hbm.at[pidx[b]], prows[b], sem_p[b])
        pltpu.async_copy(type_hbm.at[tidx[b]], trows[b], sem_t[b])

    fire(0, 0)
    fire(1, 1)

    def pair_body(cp, _):
        for b in range(2):
            c = cp * 2 + b
            pltpu.make_async_copy(word_hbm.at[widx[b]], wrows[b],
                                  sem_w[b]).wait()
            pltpu.make_async_copy(postab_hbm.at[pidx[b]], prows[b],
                                  sem_p[b]).wait()
            pltpu.make_async_copy(type_hbm.at[tidx[b]], trows[b],
                                  sem_t[b]).wait()

            @plsc.parallel_loop(0, CHUNK)
            def tok(t):
                for j in range(HID // LANES):
                    sl = pl.ds(j * LANES, LANES)
                    wrows[b][t, sl] = (wrows[b][t, sl] + prows[b][t, sl]
                                       + trows[b][t, sl])

            out_slice = out_hbm.at[pl.ds(base + c * CHUNK, CHUNK)]
            pltpu.async_copy(wrows[b], out_slice, sem_o[b])

            @pl.when(c + 2 < nch)
            def _():
                pltpu.make_async_copy(wrows[b], out_slice, sem_o[b]).wait()
                fire(c + 2, b)
        return 0

    lax.fori_loop(0, nch // 2, pair_body, 0)
    for b in range(2):
        c = nch - 2 + b
        pltpu.make_async_copy(
            wrows[b], out_hbm.at[pl.ds(base + c * CHUNK, CHUNK)],
            sem_o[b]).wait()


def kernel(input_ids, position_ids, token_type_ids, word_embeddings,
           token_type_embeddings, position_embeddings):
    b, s = input_ids.shape
    ntok = b * s
    hid = word_embeddings.shape[1]
    info = plsc.get_sparse_core_info()
    nworkers = info.num_cores * info.num_subcores
    tokens_per_worker = ntok // nworkers

    ids = input_ids.reshape(ntok).astype(jnp.int32)
    pos = position_ids.reshape(ntok).astype(jnp.int32)
    tts = token_type_ids.reshape(ntok).astype(jnp.int32)

    run = pl.kernel(
        functools.partial(_emb_body, tokens_per_worker=tokens_per_worker),
        mesh=plsc.VectorSubcoreMesh(core_axis_name="c", subcore_axis_name="s"),
        out_type=jax.ShapeDtypeStruct((ntok, hid), jnp.float32),
        scratch_types=[
            [pltpu.VMEM((CHUNK,), jnp.int32)] * 2,
            [pltpu.VMEM((CHUNK,), jnp.int32)] * 2,
            [pltpu.VMEM((CHUNK,), jnp.int32)] * 2,
            [pltpu.VMEM((CHUNK, hid), jnp.float32)] * 2,
            [pltpu.VMEM((CHUNK, hid), jnp.float32)] * 2,
            [pltpu.VMEM((CHUNK, hid), jnp.float32)] * 2,
            [pltpu.SemaphoreType.DMA] * 2,
            [pltpu.SemaphoreType.DMA] * 2,
            [pltpu.SemaphoreType.DMA] * 2,
            [pltpu.SemaphoreType.DMA] * 2,
        ],
    )
    out = run(ids, pos, tts, word_embeddings, token_type_embeddings,
              position_embeddings)
    return out.reshape(b, s, hid)


# speedup vs baseline: 1.0394x; 1.0001x over previous
import functools

import jax
import jax.numpy as jnp
from jax import lax
from jax.experimental import pallas as pl
from jax.experimental.pallas import tpu as pltpu
from jax.experimental.pallas import tpu_sc as plsc

HID = 768
LANES = 16
CHUNK = 16
NBUF = 3


def _emb_body(ids_hbm, pos_hbm, tt_hbm, word_hbm, type_hbm, postab_hbm,
              out_hbm, widx, pidx, tidx, wrows, prows, trows,
              sem_w, sem_p, sem_t, sem_o, *, tokens_per_worker):
    num_cores = 2
    wid = lax.axis_index("s") * num_cores + lax.axis_index("c")
    base = wid * tokens_per_worker
    nch = tokens_per_worker // CHUNK

    pltpu.sync_copy(ids_hbm.at[pl.ds(base, tokens_per_worker)], widx)
    pltpu.sync_copy(pos_hbm.at[pl.ds(base, tokens_per_worker)], pidx)
    pltpu.sync_copy(tt_hbm.at[pl.ds(base, tokens_per_worker)], tidx)

    def fire(c, b):
        sl = pl.ds(c * CHUNK, CHUNK)
        pltpu.async_copy(word_hbm.at[widx.at[sl]], wrows[b], sem_w[b])
        pltpu.async_copy(postab_hbm.at[pidx.at[sl]], prows[b], sem_p[b])
        pltpu.async_copy(type_hbm.at[tidx.at[sl]], trows[b], sem_t[b])

    def wait_gathers(c, b):
        sl = pl.ds(c * CHUNK, CHUNK)
        pltpu.make_async_copy(word_hbm.at[widx.at[sl]], wrows[b],
                              sem_w[b]).wait()
        pltpu.make_async_copy(postab_hbm.at[pidx.at[sl]], prows[b],
                              sem_p[b]).wait()
        pltpu.make_async_copy(type_hbm.at[tidx.at[sl]], trows[b],
                              sem_t[b]).wait()

    def out_slice(c):
        return out_hbm.at[pl.ds(base + c * CHUNK, CHUNK)]

    fire(0, 0)
    fire(1, 1)
    for c in range(nch):
        b = c % NBUF
        wait_gathers(c, b)

        @plsc.parallel_loop(0, CHUNK)
        def tok(t):
            @plsc.parallel_loop(0, HID // LANES, unroll=8)
            def col(j):
                sl = pl.ds(j * LANES, LANES)
                wrows[b][t, sl] = (wrows[b][t, sl] + prows[b][t, sl]
                                   + trows[b][t, sl])

        pltpu.async_copy(wrows[b], out_slice(c), sem_o[b])
        if c + 2 < nch:
            if c >= 1:
                pltpu.make_async_copy(wrows[(c - 1) % NBUF], out_slice(c - 1),
                                      sem_o[(c - 1) % NBUF]).wait()
            fire(c + 2, (c + 2) % NBUF)
    for c in range(max(nch - 3, 0), nch):
        pltpu.make_async_copy(wrows[c % NBUF], out_slice(c),
                              sem_o[c % NBUF]).wait()


def kernel(input_ids, position_ids, token_type_ids, word_embeddings,
           token_type_embeddings, position_embeddings):
    b, s = input_ids.shape
    ntok = b * s
    hid = word_embeddings.shape[1]
    info = plsc.get_sparse_core_info()
    nworkers = info.num_cores * info.num_subcores
    tokens_per_worker = ntok // nworkers

    ids = input_ids.reshape(ntok).astype(jnp.int32)
    pos = position_ids.reshape(ntok).astype(jnp.int32)
    tts = token_type_ids.reshape(ntok).astype(jnp.int32)

    run = pl.kernel(
        functools.partial(_emb_body, tokens_per_worker=tokens_per_worker),
        mesh=plsc.VectorSubcoreMesh(core_axis_name="c", subcore_axis_name="s"),
        out_type=jax.ShapeDtypeStruct((ntok, hid), jnp.float32),
        scratch_types=[
            pltpu.VMEM((tokens_per_worker,), jnp.int32),
            pltpu.VMEM((tokens_per_worker,), jnp.int32),
            pltpu.VMEM((tokens_per_worker,), jnp.int32),
            [pltpu.VMEM((CHUNK, hid), jnp.float32)] * NBUF,
            [pltpu.VMEM((CHUNK, hid), jnp.float32)] * NBUF,
            [pltpu.VMEM((CHUNK, hid), jnp.float32)] * NBUF,
            [pltpu.SemaphoreType.DMA] * NBUF,
            [pltpu.SemaphoreType.DMA] * NBUF,
            [pltpu.SemaphoreType.DMA] * NBUF,
            [pltpu.SemaphoreType.DMA] * NBUF,
        ],
    )
    out = run(ids, pos, tts, word_embeddings, token_type_embeddings,
              position_embeddings)
    return out.reshape(b, s, hid)


# speedup vs baseline: 4.3288x; 4.1646x over previous
import functools

import jax
import jax.numpy as jnp
from jax import lax
from jax.experimental import pallas as pl
from jax.experimental.pallas import tpu as pltpu
from jax.experimental.pallas import tpu_sc as plsc

HID = 768
LANES = 16
CHUNK = 32
NW_RING = 3
NP_RING = 2


def _take16(vec, idx):
    return lax.gather(
        vec, idx[:, None],
        lax.GatherDimensionNumbers(
            offset_dims=(), collapsed_slice_dims=(0,), start_index_map=(0,)),
        slice_sizes=(1,), mode=lax.GatherScatterMode.PROMISE_IN_BOUNDS)


def _emb_body(idx_hbm, word_hbm, type_hbm, postab_hbm, out_hbm,
              idxall, wrows, prows, ttab, sem_w, sem_p, sem_o,
              *, tokens_per_worker):
    num_cores = 2
    wid = lax.axis_index("s") * num_cores + lax.axis_index("c")
    base = wid * tokens_per_worker
    nch = tokens_per_worker // CHUNK

    pltpu.sync_copy(idx_hbm.at[pl.ds(wid * 3 * tokens_per_worker,
                                     3 * tokens_per_worker)], idxall)
    pltpu.sync_copy(type_hbm, ttab)

    def widx(c):
        return idxall.at[pl.ds(c * CHUNK, CHUNK)]

    def pidx(c):
        return idxall.at[pl.ds(tokens_per_worker + c * CHUNK, CHUNK)]

    def fire_w(c):
        pltpu.async_copy(word_hbm.at[widx(c)], wrows[c % NW_RING],
                         sem_w[c % NW_RING])

    def fire_p(c):
        pltpu.async_copy(postab_hbm.at[pidx(c)], prows[c % NP_RING],
                         sem_p[c % NP_RING])

    def wait_w(c):
        pltpu.make_async_copy(word_hbm.at[widx(c)], wrows[c % NW_RING],
                              sem_w[c % NW_RING]).wait()

    def wait_p(c):
        pltpu.make_async_copy(postab_hbm.at[pidx(c)], prows[c % NP_RING],
                              sem_p[c % NP_RING]).wait()

    def out_slice(c):
        return out_hbm.at[pl.ds(base + c * CHUNK, CHUNK)]

    def wait_out(c):
        pltpu.make_async_copy(wrows[c % NW_RING], out_slice(c),
                              sem_o[c % NW_RING]).wait()

    fire_w(0)
    fire_p(0)
    fire_w(1)
    fire_p(1)

    for c in range(nch):
        b = c % NW_RING
        q = c % NP_RING
        wait_w(c)
        wait_p(c)
        toff = 2 * tokens_per_worker + c * CHUNK

        @plsc.parallel_loop(0, CHUNK)
        def tok(t):
            g = (toff + t) & ~(LANES - 1)
            lane = (toff + t) & (LANES - 1)
            ttg = idxall[pl.ds(g, LANES)].astype(jnp.float32)
            ttf = _take16(ttg, jnp.full((LANES,), lane, jnp.int32))

            @plsc.parallel_loop(0, HID // LANES, unroll=8)
            def col(j):
                sl = pl.ds(j * LANES, LANES)
                t0 = ttab[0, sl]
                wrows[b][t, sl] = (wrows[b][t, sl] + prows[q][t, sl] + t0
                                   + ttf * (ttab[1, sl] - t0))

        pltpu.async_copy(wrows[b], out_slice(c), sem_o[b])
        if c + 2 < nch:
            fire_p(c + 2)
            if c >= 1:
                wait_out(c - 1)
            fire_w(c + 2)
    for c in range(max(nch - 3, 0), nch):
        wait_out(c)


def kernel(input_ids, position_ids, token_type_ids, word_embeddings,
           token_type_embeddings, position_embeddings):
    b, s = input_ids.shape
    ntok = b * s
    hid = word_embeddings.shape[1]
    info = plsc.get_sparse_core_info()
    nworkers = info.num_cores * info.num_subcores
    tokens_per_worker = ntok // nworkers

    idx = jnp.concatenate([
        input_ids.reshape(nworkers, tokens_per_worker),
        position_ids.reshape(nworkers, tokens_per_worker),
        token_type_ids.reshape(nworkers, tokens_per_worker),
    ], axis=1).astype(jnp.int32).reshape(-1)

    run = pl.kernel(
        functools.partial(_emb_body, tokens_per_worker=tokens_per_worker),
        mesh=plsc.VectorSubcoreMesh(core_axis_name="c", subcore_axis_name="s"),
        out_type=jax.ShapeDtypeStruct((ntok, hid), jnp.float32),
        scratch_types=[
            pltpu.VMEM((3 * tokens_per_worker,), jnp.int32),
            [pltpu.VMEM((CHUNK, hid), jnp.float32)] * NW_RING,
            [pltpu.VMEM((CHUNK, hid), jnp.float32)] * NP_RING,
            pltpu.VMEM((2, hid), jnp.float32),
            [pltpu.SemaphoreType.DMA] * NW_RING,
            [pltpu.SemaphoreType.DMA] * NP_RING,
            [pltpu.SemaphoreType.DMA] * NW_RING,
        ],
    )
    out = run(idx, word_embeddings, token_type_embeddings,
              position_embeddings)
    return out.reshape(b, s, hid)
